# baseline (speedup 1.0000x reference)
import functools

import jax
import jax.numpy as jnp
from jax import lax
from jax.experimental import pallas as pl
from jax.experimental.pallas import tpu as pltpu
from jax.experimental.pallas import tpu_sc as plsc

N = 8192
L = 8
D_IN = 896
D_HID = 1536
D_OUT = 768

BLK = 256
NB = N // BLK + L
NPAD = NB * BLK

NC = 2
NS = 16
NW = NC * NS
TOK_W = N // NW
SCAT_CHUNK = 64
GATH_CHUNK = 64

_mesh = plsc.VectorSubcoreMesh(core_axis_name="c", subcore_axis_name="s")


_NCH_S = TOK_W // SCAT_CHUNK
_NCH_G = TOK_W // GATH_CHUNK


@functools.partial(
    pl.kernel,
    mesh=_mesh,
    out_type=jax.ShapeDtypeStruct((NPAD, D_IN), jnp.float32),
    scratch_types=[
        pltpu.VMEM((SCAT_CHUNK,), jnp.int32),
        pltpu.VMEM((SCAT_CHUNK,), jnp.int32),
        pltpu.VMEM((SCAT_CHUNK, D_IN), jnp.float32),
        pltpu.VMEM((SCAT_CHUNK, D_IN), jnp.float32),
        pltpu.SemaphoreType.DMA,
        pltpu.SemaphoreType.DMA,
        pltpu.SemaphoreType.DMA,
        pltpu.SemaphoreType.DMA,
    ],
)
def _sc_scatter(x_hbm, dest_hbm, xpad_hbm, idx0, idx1, rows0, rows1,
                ls0, ls1, ss0, ss1):
    wid = lax.axis_index("s") * NC + lax.axis_index("c")
    base = wid * TOK_W
    idxs, rows = (idx0, idx1), (rows0, rows1)
    lsems, ssems = (ls0, ls1), (ss0, ss1)

    def loads(i):
        off = base + i * SCAT_CHUNK
        a = pltpu.async_copy(dest_hbm.at[pl.ds(off, SCAT_CHUNK)],
                             idxs[i % 2], lsems[i % 2])
        b = pltpu.async_copy(x_hbm.at[pl.ds(off, SCAT_CHUNK)],
                             rows[i % 2], lsems[i % 2])
        return a, b

    pend_l = {0: loads(0)}
    pend_s = {}
    for i in range(_NCH_S):
        if i >= 1:
            pend_s.pop(i - 1).wait()
        if i + 1 < _NCH_S:
            pend_l[i + 1] = loads(i + 1)
        la, lb = pend_l.pop(i)
        la.wait()
        lb.wait()
        pend_s[i] = pltpu.async_copy(rows[i % 2], xpad_hbm.at[idxs[i % 2]],
                                     ssems[i % 2])
    pend_s.pop(_NCH_S - 1).wait()


@functools.partial(
    pl.kernel,
    mesh=_mesh,
    out_type=jax.ShapeDtypeStruct((N, D_OUT), jnp.float32),
    scratch_types=[
        pltpu.VMEM((GATH_CHUNK,), jnp.int32),
        pltpu.VMEM((GATH_CHUNK,), jnp.int32),
        pltpu.VMEM((GATH_CHUNK, D_OUT), jnp.float32),
        pltpu.VMEM((GATH_CHUNK, D_OUT), jnp.float32),
        pltpu.SemaphoreType.DMA,
        pltpu.SemaphoreType.DMA,
        pltpu.SemaphoreType.DMA,
        pltpu.SemaphoreType.DMA,
        pltpu.SemaphoreType.DMA,
        pltpu.SemaphoreType.DMA,
    ],
)
def _sc_gather(ypad_hbm, dest_hbm, out_hbm, idx0, idx1, rows0, rows1,
               ls0, ls1, gs0, gs1, ss0, ss1):
    wid = lax.axis_index("s") * NC + lax.axis_index("c")
    base = wid * TOK_W
    idxs, rows = (idx0, idx1), (rows0, rows1)
    lsems, gsems, ssems = (ls0, ls1), (gs0, gs1), (ss0, ss1)

    def idx_load(i):
        off = base + i * GATH_CHUNK
        return pltpu.async_copy(dest_hbm.at[pl.ds(off, GATH_CHUNK)],
                                idxs[i % 2], lsems[i % 2])

    pend_idx = {0: idx_load(0)}
    pend_g = {}
    pend_st = {}
    for i in range(_NCH_G):
        if i >= 1:
            pend_g.pop(i - 1).wait()
            off_p = base + (i - 1) * GATH_CHUNK
            pend_st[i - 1] = pltpu.async_copy(
                rows[(i - 1) % 2], out_hbm.at[pl.ds(off_p, GATH_CHUNK)],
                ssems[(i - 1) % 2])
        if i + 1 < _NCH_G:
            pend_idx[i + 1] = idx_load(i + 1)
        if i >= 2:
            pend_st.pop(i - 2).wait()
        pend_idx.pop(i).wait()
        pend_g[i] = pltpu.async_copy(ypad_hbm.at[idxs[i % 2]], rows[i % 2],
                                     gsems[i % 2])
    last = _NCH_G - 1
    pend_g.pop(last).wait()
    off_l = base + last * GATH_CHUNK
    pend_st[last] = pltpu.async_copy(rows[last % 2],
                                     out_hbm.at[pl.ds(off_l, GATH_CHUNK)],
                                     ssems[last % 2])
    for k in list(pend_st):
        pend_st.pop(k).wait()


_O_FIRST = NB + 1
_O_PAR = 2 * NB + 1
_O_NXT = 3 * NB + 1


def _mlp_body(bp_ref, x_ref, w1_hbm, b1_ref, w2_hbm, b2_ref, o_ref,
              w1buf, w2buf, w1bf, w2bf, sems):
    b = pl.program_id(0)
    nact = bp_ref[NB]
    first = bp_ref[_O_FIRST + b]
    cur = bp_ref[_O_PAR + b]
    nxt = bp_ref[_O_NXT + b]

    def w_copies(plane, slot):
        c1 = pltpu.make_async_copy(w1_hbm.at[plane], w1buf.at[slot],
                                   sems.at[slot])
        c2 = pltpu.make_async_copy(w2_hbm.at[plane], w2buf.at[slot],
                                   sems.at[slot])
        return c1, c2

    @pl.when(b == 0)
    def _():
        c1, c2 = w_copies(bp_ref[0], 0)
        c1.start()
        c2.start()

    @pl.when(first == 1)
    def _():
        c1, c2 = w_copies(bp_ref[b], cur)
        c1.wait()
        c2.wait()

        @pl.when(nxt >= 0)
        def _():
            n1, n2 = w_copies(nxt, 1 - cur)
            n1.start()
            n2.start()

        w1bf[...] = w1buf[cur].astype(jnp.bfloat16)
        w2bf[...] = w2buf[cur].astype(jnp.bfloat16)

    @pl.when(b < nact)
    def _():
        x = x_ref[...].astype(jnp.bfloat16)
        h = lax.dot_general(x, w1bf[...], (((1,), (1,)), ((), ())),
                            preferred_element_type=jnp.float32)
        h = h + b1_ref[0]
        h = 0.5 * h * (1.0 + lax.erf(h * (2.0 ** -0.5)))
        o = lax.dot_general(h.astype(jnp.bfloat16), w2bf[...],
                            (((1,), (1,)), ((), ())),
                            preferred_element_type=jnp.float32)
        o_ref[...] = o + b2_ref[0]


_mlp_call = pl.pallas_call(
    _mlp_body,
    grid_spec=pltpu.PrefetchScalarGridSpec(
        num_scalar_prefetch=1,
        grid=(NB,),
        in_specs=[
            pl.BlockSpec((BLK, D_IN), lambda b, bp: (b, 0)),
            pl.BlockSpec(memory_space=pltpu.MemorySpace.HBM),
            pl.BlockSpec((1, 1, D_HID), lambda b, bp: (bp[b], 0, 0)),
            pl.BlockSpec(memory_space=pltpu.MemorySpace.HBM),
            pl.BlockSpec((1, 1, D_OUT), lambda b, bp: (bp[b], 0, 0)),
        ],
        out_specs=pl.BlockSpec((BLK, D_OUT), lambda b, bp: (b, 0)),
        scratch_shapes=[
            pltpu.VMEM((2, D_HID, D_IN), jnp.float32),
            pltpu.VMEM((2, D_OUT, D_HID), jnp.float32),
            pltpu.VMEM((D_HID, D_IN), jnp.bfloat16),
            pltpu.VMEM((D_OUT, D_HID), jnp.bfloat16),
            pltpu.SemaphoreType.DMA((2,)),
        ],
    ),
    out_shape=jax.ShapeDtypeStruct((NPAD, D_OUT), jnp.float32),
)


def kernel(x, plane_idx, W1, b1, W2, b2):
    pid = plane_idx.astype(jnp.int32)

    onehot = (pid[:, None] == jnp.arange(L, dtype=jnp.int32)).astype(jnp.int32)
    csum = jnp.cumsum(onehot, axis=0)
    cnt = csum[-1]
    nblk_p = (cnt + BLK - 1) // BLK
    blk_start = jnp.concatenate(
        [jnp.zeros((1,), jnp.int32), jnp.cumsum(nblk_p)[:-1].astype(jnp.int32)])
    dest = jnp.sum(onehot * (blk_start[None, :] * BLK + csum - 1),
                   axis=1).astype(jnp.int32)
    nact = jnp.sum(nblk_p).astype(jnp.int32)

    barange = jnp.arange(NB, dtype=jnp.int32)
    blk_end = (blk_start + nblk_p).astype(jnp.int32)
    blk_plane = jnp.sum((barange[:, None] >= blk_end[None, :]).astype(jnp.int32),
                        axis=1)
    last_plane = jnp.max(jnp.where(cnt > 0, jnp.arange(L, dtype=jnp.int32), 0))
    blk_plane = jnp.where(barange < nact, blk_plane, last_plane).astype(jnp.int32)

    prev_plane = jnp.concatenate([jnp.full((1,), -1, jnp.int32), blk_plane[:-1]])
    first = ((blk_plane != prev_plane) & (barange < nact)).astype(jnp.int32)
    run_idx = jnp.cumsum(first) - 1
    parity = (run_idx % 2).astype(jnp.int32)
    planes = jnp.arange(L, dtype=jnp.int32)
    later = (planes[None, :] > blk_plane[:, None]) & (cnt[None, :] > 0)
    nxt = jnp.min(jnp.where(later, planes[None, :], L), axis=1).astype(jnp.int32)
    nxt = jnp.where(nxt >= L, -1, nxt)

    bp_arr = jnp.concatenate([blk_plane, nact[None], first, parity, nxt])

    x_pad = _sc_scatter(x, dest)
    y_pad = _mlp_call(bp_arr, x_pad, W1,
                      b1.reshape(L, 1, D_HID), W2, b2.reshape(L, 1, D_OUT))
    out = _sc_gather(y_pad, dest)
    return out


# speedup vs baseline: 1.0267x; 1.0267x over previous
import functools

import jax
import jax.numpy as jnp
from jax import lax
from jax.experimental import pallas as pl
from jax.experimental.pallas import tpu as pltpu
from jax.experimental.pallas import tpu_sc as plsc

N = 8192
L = 8
D_IN = 896
D_HID = 1536
D_OUT = 768

BLK = 256
NB = N // BLK + L
NPAD = NB * BLK

NC = 2
NS = 16
NW = NC * NS
TOK_W = N // NW
SCAT_CHUNK = 64
GATH_CHUNK = 64

_mesh = plsc.VectorSubcoreMesh(core_axis_name="c", subcore_axis_name="s")


_NCH_S = TOK_W // SCAT_CHUNK
_NCH_G = TOK_W // GATH_CHUNK


@functools.partial(
    pl.kernel,
    mesh=_mesh,
    out_type=jax.ShapeDtypeStruct((NPAD, D_IN), jnp.float32),
    scratch_types=[
        pltpu.VMEM((SCAT_CHUNK,), jnp.int32),
        pltpu.VMEM((SCAT_CHUNK,), jnp.int32),
        pltpu.VMEM((SCAT_CHUNK, D_IN), jnp.float32),
        pltpu.VMEM((SCAT_CHUNK, D_IN), jnp.float32),
        pltpu.SemaphoreType.DMA,
        pltpu.SemaphoreType.DMA,
        pltpu.SemaphoreType.DMA,
        pltpu.SemaphoreType.DMA,
    ],
)
def _sc_scatter(x_hbm, dest_hbm, xpad_hbm, idx0, idx1, rows0, rows1,
                ls0, ls1, ss0, ss1):
    wid = lax.axis_index("s") * NC + lax.axis_index("c")
    base = wid * TOK_W
    idxs, rows = (idx0, idx1), (rows0, rows1)
    lsems, ssems = (ls0, ls1), (ss0, ss1)

    def loads(i):
        off = base + i * SCAT_CHUNK
        a = pltpu.async_copy(dest_hbm.at[pl.ds(off, SCAT_CHUNK)],
                             idxs[i % 2], lsems[i % 2])
        b = pltpu.async_copy(x_hbm.at[pl.ds(off, SCAT_CHUNK)],
                             rows[i % 2], lsems[i % 2])
        return a, b

    pend_l = {0: loads(0)}
    pend_s = {}
    for i in range(_NCH_S):
        if i >= 1:
            pend_s.pop(i - 1).wait()
        if i + 1 < _NCH_S:
            pend_l[i + 1] = loads(i + 1)
        la, lb = pend_l.pop(i)
        la.wait()
        lb.wait()
        pend_s[i] = pltpu.async_copy(rows[i % 2], xpad_hbm.at[idxs[i % 2]],
                                     ssems[i % 2])
    pend_s.pop(_NCH_S - 1).wait()


@functools.partial(
    pl.kernel,
    mesh=_mesh,
    out_type=jax.ShapeDtypeStruct((N, D_OUT), jnp.float32),
    scratch_types=[
        pltpu.VMEM((GATH_CHUNK,), jnp.int32),
        pltpu.VMEM((GATH_CHUNK,), jnp.int32),
        pltpu.VMEM((GATH_CHUNK, D_OUT), jnp.float32),
        pltpu.VMEM((GATH_CHUNK, D_OUT), jnp.float32),
        pltpu.SemaphoreType.DMA,
        pltpu.SemaphoreType.DMA,
        pltpu.SemaphoreType.DMA,
        pltpu.SemaphoreType.DMA,
        pltpu.SemaphoreType.DMA,
        pltpu.SemaphoreType.DMA,
    ],
)
def _sc_gather(ypad_hbm, dest_hbm, out_hbm, idx0, idx1, rows0, rows1,
               ls0, ls1, gs0, gs1, ss0, ss1):
    wid = lax.axis_index("s") * NC + lax.axis_index("c")
    base = wid * TOK_W
    idxs, rows = (idx0, idx1), (rows0, rows1)
    lsems, gsems, ssems = (ls0, ls1), (gs0, gs1), (ss0, ss1)

    def idx_load(i):
        off = base + i * GATH_CHUNK
        return pltpu.async_copy(dest_hbm.at[pl.ds(off, GATH_CHUNK)],
                                idxs[i % 2], lsems[i % 2])

    pend_idx = {0: idx_load(0)}
    pend_g = {}
    pend_st = {}
    for i in range(_NCH_G):
        if i >= 1:
            pend_g.pop(i - 1).wait()
            off_p = base + (i - 1) * GATH_CHUNK
            pend_st[i - 1] = pltpu.async_copy(
                rows[(i - 1) % 2], out_hbm.at[pl.ds(off_p, GATH_CHUNK)],
                ssems[(i - 1) % 2])
        if i + 1 < _NCH_G:
            pend_idx[i + 1] = idx_load(i + 1)
        if i >= 2:
            pend_st.pop(i - 2).wait()
        pend_idx.pop(i).wait()
        pend_g[i] = pltpu.async_copy(ypad_hbm.at[idxs[i % 2]], rows[i % 2],
                                     gsems[i % 2])
    last = _NCH_G - 1
    pend_g.pop(last).wait()
    off_l = base + last * GATH_CHUNK
    pend_st[last] = pltpu.async_copy(rows[last % 2],
                                     out_hbm.at[pl.ds(off_l, GATH_CHUNK)],
                                     ssems[last % 2])
    for k in list(pend_st):
        pend_st.pop(k).wait()


_O_FIRST = NB + 1
_O_PAR = 2 * NB + 1
_O_NXT = 3 * NB + 1


def _mlp_body(bp_ref, x_ref, w1_hbm, b1_ref, w2_hbm, b2_ref, o_ref,
              w1buf, w2buf, w1bf, w2bf, sems):
    b = pl.program_id(0)
    nact = bp_ref[NB]
    first = bp_ref[_O_FIRST + b]
    cur = bp_ref[_O_PAR + b]
    nxt = bp_ref[_O_NXT + b]

    def w_copies(plane, slot):
        c1 = pltpu.make_async_copy(w1_hbm.at[plane], w1buf.at[slot],
                                   sems.at[slot])
        c2 = pltpu.make_async_copy(w2_hbm.at[plane], w2buf.at[slot],
                                   sems.at[slot])
        return c1, c2

    @pl.when(b == 0)
    def _():
        c1, c2 = w_copies(bp_ref[0], 0)
        c1.start()
        c2.start()

    @pl.when(first == 1)
    def _():
        c1, c2 = w_copies(bp_ref[b], cur)
        c1.wait()
        c2.wait()

        @pl.when(nxt >= 0)
        def _():
            n1, n2 = w_copies(nxt, 1 - cur)
            n1.start()
            n2.start()

        w1bf[...] = w1buf[cur].astype(jnp.bfloat16)
        w2bf[...] = w2buf[cur].astype(jnp.bfloat16)

    @pl.when(b < nact)
    def _():
        x = x_ref[...].astype(jnp.bfloat16)
        h = lax.dot_general(x, w1bf[...], (((1,), (1,)), ((), ())),
                            preferred_element_type=jnp.float32)
        h = h + b1_ref[0]
        h = 0.5 * h * (1.0 + lax.erf(h * (2.0 ** -0.5)))
        o = lax.dot_general(h.astype(jnp.bfloat16), w2bf[...],
                            (((1,), (1,)), ((), ())),
                            preferred_element_type=jnp.float32)
        o_ref[...] = o + b2_ref[0]


_mlp_call = pl.pallas_call(
    _mlp_body,
    grid_spec=pltpu.PrefetchScalarGridSpec(
        num_scalar_prefetch=1,
        grid=(NB,),
        in_specs=[
            pl.BlockSpec((BLK, D_IN),
                         lambda b, bp: (jnp.minimum(b, bp[NB] - 1), 0)),
            pl.BlockSpec(memory_space=pltpu.MemorySpace.HBM),
            pl.BlockSpec((1, 1, D_HID), lambda b, bp: (bp[b], 0, 0)),
            pl.BlockSpec(memory_space=pltpu.MemorySpace.HBM),
            pl.BlockSpec((1, 1, D_OUT), lambda b, bp: (bp[b], 0, 0)),
        ],
        out_specs=pl.BlockSpec(
            (BLK, D_OUT),
            lambda b, bp: (jnp.where(b < bp[NB], b, NB), 0)),
        scratch_shapes=[
            pltpu.VMEM((2, D_HID, D_IN), jnp.float32),
            pltpu.VMEM((2, D_OUT, D_HID), jnp.float32),
            pltpu.VMEM((D_HID, D_IN), jnp.bfloat16),
            pltpu.VMEM((D_OUT, D_HID), jnp.bfloat16),
            pltpu.SemaphoreType.DMA((2,)),
        ],
    ),
    out_shape=jax.ShapeDtypeStruct((NPAD + BLK, D_OUT), jnp.float32),
)


def kernel(x, plane_idx, W1, b1, W2, b2):
    pid = plane_idx.astype(jnp.int32)

    onehot = (pid[:, None] == jnp.arange(L, dtype=jnp.int32)).astype(jnp.int32)
    csum = jnp.cumsum(onehot, axis=0)
    cnt = csum[-1]
    nblk_p = (cnt + BLK - 1) // BLK
    blk_start = jnp.concatenate(
        [jnp.zeros((1,), jnp.int32), jnp.cumsum(nblk_p)[:-1].astype(jnp.int32)])
    dest = jnp.sum(onehot * (blk_start[None, :] * BLK + csum - 1),
                   axis=1).astype(jnp.int32)
    nact = jnp.sum(nblk_p).astype(jnp.int32)

    barange = jnp.arange(NB, dtype=jnp.int32)
    blk_end = (blk_start + nblk_p).astype(jnp.int32)
    blk_plane = jnp.sum((barange[:, None] >= blk_end[None, :]).astype(jnp.int32),
                        axis=1)
    last_plane = jnp.max(jnp.where(cnt > 0, jnp.arange(L, dtype=jnp.int32), 0))
    blk_plane = jnp.where(barange < nact, blk_plane, last_plane).astype(jnp.int32)

    prev_plane = jnp.concatenate([jnp.full((1,), -1, jnp.int32), blk_plane[:-1]])
    first = ((blk_plane != prev_plane) & (barange < nact)).astype(jnp.int32)
    run_idx = jnp.cumsum(first) - 1
    parity = (run_idx % 2).astype(jnp.int32)
    planes = jnp.arange(L, dtype=jnp.int32)
    later = (planes[None, :] > blk_plane[:, None]) & (cnt[None, :] > 0)
    nxt = jnp.min(jnp.where(later, planes[None, :], L), axis=1).astype(jnp.int32)
    nxt = jnp.where(nxt >= L, -1, nxt)

    bp_arr = jnp.concatenate([blk_plane, nact[None], first, parity, nxt])

    x_pad = _sc_scatter(x, dest)
    y_pad = _mlp_call(bp_arr, x_pad, W1,
                      b1.reshape(L, 1, D_HID), W2, b2.reshape(L, 1, D_OUT))
    out = _sc_gather(y_pad, dest)
    return out
